# baseline (speedup 1.0000x reference)
import functools

import jax
import jax.numpy as jnp
from jax import lax
from jax.experimental import pallas as pl
from jax.experimental.pallas import tpu as pltpu
from jax.experimental.pallas import tpu_sc as plsc

N = 10000
D = 128
NC = 2
NS = 16
NW = NC * NS
K = 128

ZROWS = 632
ACC_ROWS = NS * ZROWS

DEG_ROWS = 10240
DEG_Z = DEG_ROWS // NS


def _mesh():
    return plsc.VectorSubcoreMesh(core_axis_name="c", subcore_axis_name="s")


def _make_deg_kernel(ch, e):
    @functools.partial(
        pl.kernel,
        out_type=jax.ShapeDtypeStruct((NC, DEG_ROWS), jnp.float32),
        mesh=_mesh(),
        scratch_types=[
            pltpu.VMEM((ch, K), jnp.int32),
            pltpu.VMEM((K,), jnp.float32),
            pltpu.VMEM_SHARED((DEG_ROWS,), jnp.float32),
        ],
    )
    def deg_kernel(dst_hbm, zeros_hbm, out_hbm, dst_v, ones_v, acc):
        c = lax.axis_index("c")
        s = lax.axis_index("s")
        w = s * NC + c
        n_w = jnp.minimum(ch, jnp.maximum(0, (e - w * (ch * K) + K - 1) // K))
        pltpu.sync_copy(dst_hbm.at[w], dst_v)
        for i in range(K // 16):
            ones_v[pl.ds(i * 16, 16)] = jnp.ones((16,), jnp.float32)
        pltpu.sync_copy(zeros_hbm, acc.at[pl.ds(s * DEG_Z, DEG_Z)])
        plsc.subcore_barrier()

        def body(j, carry):
            pltpu.sync_copy(ones_v, acc.at[dst_v.at[j]], add=True)
            return carry

        lax.fori_loop(0, n_w, body, 0)
        plsc.subcore_barrier()
        pltpu.sync_copy(acc.at[pl.ds(s * DEG_Z, DEG_Z)],
                        out_hbm.at[c].at[pl.ds(s * DEG_Z, DEG_Z)])

    return deg_kernel


W_CH = 16


def _make_scatter_kernel(ch, e):
    @functools.partial(
        pl.kernel,
        out_type=jax.ShapeDtypeStruct((NC, ACC_ROWS, D), jnp.float32),
        mesh=_mesh(),
        scratch_types=[
            pltpu.VMEM((W_CH, K), jnp.int32),
            pltpu.VMEM((W_CH, K), jnp.int32),
            pltpu.VMEM((W_CH, K), jnp.int32),
            pltpu.VMEM((W_CH, K), jnp.int32),
            pltpu.VMEM((K, D), jnp.float32),
            pltpu.VMEM((K, D), jnp.float32),
            pltpu.VMEM((40, D), jnp.float32),
            pltpu.VMEM_SHARED((ACC_ROWS, D), jnp.float32),
            pltpu.SemaphoreType.DMA,
            pltpu.SemaphoreType.DMA,
            pltpu.SemaphoreType.DMA,
            pltpu.SemaphoreType.DMA,
        ],
    )
    def scatter_kernel(g_hbm, src_hbm, dst_hbm, out_hbm,
                       src0, dst0, src1, dst1, buf_a, buf_b, zbuf, acc,
                       sem_a, sem_b, sem_i0, sem_i1):
        c = lax.axis_index("c")
        s = lax.axis_index("s")
        w = s * NC + c
        n_w = jnp.minimum(ch, jnp.maximum(0, (e - w * (ch * K) + K - 1) // K))
        n_win = (n_w + W_CH - 1) // W_CH

        def load_idx(wi, sv, dv, sem):
            base = wi * W_CH
            pltpu.async_copy(src_hbm.at[w].at[pl.ds(base, W_CH)], sv, sem)
            pltpu.async_copy(dst_hbm.at[w].at[pl.ds(base, W_CH)], dv, sem)

        def wait_idx(sv, dv, sem):
            pltpu.make_async_copy(src_hbm.at[w].at[pl.ds(0, W_CH)], sv,
                                  sem).wait()
            pltpu.make_async_copy(dst_hbm.at[w].at[pl.ds(0, W_CH)], dv,
                                  sem).wait()

        def process(wi, sv, dv):
            m = jnp.minimum(W_CH, n_w - wi * W_CH)
            pltpu.async_copy(g_hbm.at[sv.at[0]], buf_a, sem_a)

            def pair(t, carry):
                j0 = 2 * t
                j1 = j0 + 1

                @pl.when(j1 < m)
                def _():
                    pltpu.async_copy(g_hbm.at[sv.at[j1]], buf_b, sem_b)

                pltpu.make_async_copy(g_hbm.at[sv.at[j0]], buf_a, sem_a).wait()
                pltpu.sync_copy(buf_a, acc.at[dv.at[j0]], add=True)

                @pl.when(j0 + 2 < m)
                def _():
                    pltpu.async_copy(g_hbm.at[sv.at[j0 + 2]], buf_a, sem_a)

                @pl.when(j1 < m)
                def _():
                    pltpu.make_async_copy(g_hbm.at[sv.at[j1]], buf_b,
                                          sem_b).wait()
                    pltpu.sync_copy(buf_b, acc.at[dv.at[j1]], add=True)

                return carry

            lax.fori_loop(0, (m + 1) // 2, pair, 0)

        @pl.when(n_win > 0)
        def _():
            load_idx(0, src0, dst0, sem_i0)
        for i in range(40):
            for k in range(D // 16):
                zbuf[i, pl.ds(k * 16, 16)] = jnp.zeros((16,), jnp.float32)
        for r in range(15):
            pltpu.sync_copy(zbuf, acc.at[pl.ds(s * ZROWS + r * 40, 40)])
        pltpu.sync_copy(zbuf.at[pl.ds(0, 32)],
                        acc.at[pl.ds(s * ZROWS + 600, 32)])
        plsc.subcore_barrier()

        def win_pair(t, carry):
            i0 = 2 * t
            i1 = i0 + 1

            @pl.when(i1 < n_win)
            def _():
                load_idx(i1, src1, dst1, sem_i1)

            wait_idx(src0, dst0, sem_i0)
            process(i0, src0, dst0)

            @pl.when(i0 + 2 < n_win)
            def _():
                load_idx(i0 + 2, src0, dst0, sem_i0)

            @pl.when(i1 < n_win)
            def _():
                wait_idx(src1, dst1, sem_i1)
                process(i1, src1, dst1)

            return carry

        lax.fori_loop(0, (n_win + 1) // 2, win_pair, 0)
        plsc.subcore_barrier()
        pltpu.sync_copy(acc.at[pl.ds(s * ZROWS, ZROWS)],
                        out_hbm.at[c].at[pl.ds(s * ZROWS, ZROWS)])

    return scatter_kernel


BM = 2000


def _mm_tc(x_ref, w_ref, h_ref):
    h_ref[...] = x_ref[...] @ w_ref[...]


def _scale_tc(h_ref, degp_ref, g_ref):
    deg = degp_ref[0] + degp_ref[1] + 1.0
    d = lax.rsqrt(deg)
    g_ref[...] = h_ref[...] * d


def _mid_tc(s_ref, g_ref, w_ref, b_ref, degp_ref, o_ref):
    deg = degp_ref[0] + degp_ref[1] + 1.0
    d = lax.rsqrt(deg)
    z = d * (s_ref[0] + s_ref[1] + g_ref[...]) + b_ref[...]
    h = jnp.maximum(z, 0.0) @ w_ref[...]
    o_ref[...] = h * d


def _last_tc(s_ref, g_ref, b_ref, degp_ref, o_ref):
    deg = degp_ref[0] + degp_ref[1] + 1.0
    d = lax.rsqrt(deg)
    o_ref[...] = d * (s_ref[0] + s_ref[1] + g_ref[...]) + b_ref[...]


def _row_grid():
    return N // BM


_SPEC_S = pl.BlockSpec((2, BM, D), lambda i: (0, i, 0))
_SPEC_ROWS = pl.BlockSpec((BM, D), lambda i: (i, 0))
_SPEC_W = pl.BlockSpec((D, D), lambda i: (0, 0))
_SPEC_B = pl.BlockSpec((1, D), lambda i: (0, 0))
_SPEC_DEG = pl.BlockSpec((2, BM, 1), lambda i: (0, i, 0))


def kernel(x, edge_index, W1, b1, W2, b2, W3, b3):
    src = edge_index[0]
    dst = edge_index[1]
    e = src.shape[0]
    ch = -(-e // (NW * K))
    ch = ((ch + W_CH - 1) // W_CH) * W_CH
    e_pad = NW * ch * K
    pad = e_pad - e
    pad_dst = N + (jnp.arange(pad, dtype=jnp.int32) % (ACC_ROWS - N))
    src_p = jnp.concatenate([src, jnp.zeros((pad,), jnp.int32)]).reshape(NW, ch, K)
    dst_p = jnp.concatenate([dst, pad_dst]).reshape(NW, ch, K)

    zeros_deg = jnp.zeros((DEG_Z,), jnp.float32)

    scatter = _make_scatter_kernel(ch, e)

    mm = pl.pallas_call(
        _mm_tc,
        grid=(_row_grid(),),
        in_specs=[_SPEC_ROWS, _SPEC_W],
        out_specs=_SPEC_ROWS,
        out_shape=jax.ShapeDtypeStruct((N, D), jnp.float32),
    )
    scale = pl.pallas_call(
        _scale_tc,
        grid=(_row_grid(),),
        in_specs=[_SPEC_ROWS, _SPEC_DEG],
        out_specs=_SPEC_ROWS,
        out_shape=jax.ShapeDtypeStruct((N, D), jnp.float32),
    )
    mid = pl.pallas_call(
        _mid_tc,
        grid=(_row_grid(),),
        in_specs=[_SPEC_S, _SPEC_ROWS, _SPEC_W, _SPEC_B, _SPEC_DEG],
        out_specs=_SPEC_ROWS,
        out_shape=jax.ShapeDtypeStruct((N, D), jnp.float32),
    )
    last = pl.pallas_call(
        _last_tc,
        grid=(_row_grid(),),
        in_specs=[_SPEC_S, _SPEC_ROWS, _SPEC_B, _SPEC_DEG],
        out_specs=_SPEC_ROWS,
        out_shape=jax.ShapeDtypeStruct((N, D), jnp.float32),
    )

    h1 = mm(x, W1)
    degp = _make_deg_kernel(ch, e)(dst_p, zeros_deg)
    degp = degp[:, :N, None]
    g1 = scale(h1, degp)
    s1 = scatter(g1, src_p, dst_p)
    g2 = mid(s1, g1, W2, b1.reshape(1, D), degp)
    s2 = scatter(g2, src_p, dst_p)
    g3 = mid(s2, g2, W3, b2.reshape(1, D), degp)
    s3 = scatter(g3, src_p, dst_p)
    return last(s3, g3, b3.reshape(1, D), degp)


# speedup vs baseline: 1.0358x; 1.0358x over previous
import functools

import jax
import jax.numpy as jnp
from jax import lax
from jax.experimental import pallas as pl
from jax.experimental.pallas import tpu as pltpu
from jax.experimental.pallas import tpu_sc as plsc

N = 10000
D = 128
NC = 2
NS = 16
NW = NC * NS
K = 128

ZROWS = 632
ACC_ROWS = NS * ZROWS

DEG_ROWS = 10240
DEG_Z = DEG_ROWS // NS


def _mesh():
    return plsc.VectorSubcoreMesh(core_axis_name="c", subcore_axis_name="s")


def _make_deg_kernel(ch, e):
    @functools.partial(
        pl.kernel,
        out_type=jax.ShapeDtypeStruct((NC, DEG_ROWS), jnp.float32),
        mesh=_mesh(),
        scratch_types=[
            pltpu.VMEM((ch, K), jnp.int32),
            pltpu.VMEM((K,), jnp.float32),
            pltpu.VMEM_SHARED((DEG_ROWS,), jnp.float32),
        ],
    )
    def deg_kernel(dst_hbm, zeros_hbm, out_hbm, dst_v, ones_v, acc):
        c = lax.axis_index("c")
        s = lax.axis_index("s")
        w = s * NC + c
        n_w = jnp.minimum(ch, jnp.maximum(0, (e - w * (ch * K) + K - 1) // K))
        pltpu.sync_copy(dst_hbm.at[w], dst_v)
        for i in range(K // 16):
            ones_v[pl.ds(i * 16, 16)] = jnp.ones((16,), jnp.float32)
        pltpu.sync_copy(zeros_hbm, acc.at[pl.ds(s * DEG_Z, DEG_Z)])
        plsc.subcore_barrier()

        def body(j, carry):
            pltpu.sync_copy(ones_v, acc.at[dst_v.at[j]], add=True)
            return carry

        lax.fori_loop(0, n_w, body, 0)
        plsc.subcore_barrier()
        pltpu.sync_copy(acc.at[pl.ds(s * DEG_Z, DEG_Z)],
                        out_hbm.at[c].at[pl.ds(s * DEG_Z, DEG_Z)])

    return deg_kernel


W_CH = 16


def _make_scatter_kernel(ch, e):
    @functools.partial(
        pl.kernel,
        out_type=jax.ShapeDtypeStruct((NC, ACC_ROWS, D), jnp.float32),
        mesh=_mesh(),
        scratch_types=[
            pltpu.VMEM((W_CH, K), jnp.int32),
            pltpu.VMEM((W_CH, K), jnp.int32),
            pltpu.VMEM((W_CH, K), jnp.int32),
            pltpu.VMEM((W_CH, K), jnp.int32),
            pltpu.VMEM((K, D), jnp.float32),
            pltpu.VMEM((K, D), jnp.float32),
            pltpu.VMEM((40, D), jnp.float32),
            pltpu.VMEM_SHARED((ACC_ROWS, D), jnp.float32),
            pltpu.SemaphoreType.DMA,
            pltpu.SemaphoreType.DMA,
            pltpu.SemaphoreType.DMA,
            pltpu.SemaphoreType.DMA,
        ],
    )
    def scatter_kernel(g_hbm, src_hbm, dst_hbm, out_hbm,
                       src0, dst0, src1, dst1, buf_a, buf_b, zbuf, acc,
                       sem_a, sem_b, sem_i0, sem_i1):
        c = lax.axis_index("c")
        s = lax.axis_index("s")
        w = s * NC + c
        n_w = jnp.minimum(ch, jnp.maximum(0, (e - w * (ch * K) + K - 1) // K))
        n_win = (n_w + W_CH - 1) // W_CH

        def load_idx(wi, sv, dv, sem):
            base = wi * W_CH
            pltpu.async_copy(src_hbm.at[w].at[pl.ds(base, W_CH)], sv, sem)
            pltpu.async_copy(dst_hbm.at[w].at[pl.ds(base, W_CH)], dv, sem)

        def wait_idx(sv, dv, sem):
            pltpu.make_async_copy(src_hbm.at[w].at[pl.ds(0, W_CH)], sv,
                                  sem).wait()
            pltpu.make_async_copy(dst_hbm.at[w].at[pl.ds(0, W_CH)], dv,
                                  sem).wait()

        sets = ((src0, dst0, sem_i0), (src1, dst1, sem_i1))

        @pl.when(n_w > 0)
        def _():
            load_idx(0, src0, dst0, sem_i0)
        for i in range(40):
            for k in range(D // 16):
                zbuf[i, pl.ds(k * 16, 16)] = jnp.zeros((16,), jnp.float32)
        for r in range(15):
            pltpu.sync_copy(zbuf, acc.at[pl.ds(s * ZROWS + r * 40, 40)])
        pltpu.sync_copy(zbuf.at[pl.ds(0, 32)],
                        acc.at[pl.ds(s * ZROWS + 600, 32)])

        @pl.when(n_w > 0)
        def _():
            wait_idx(src0, dst0, sem_i0)
            pltpu.async_copy(g_hbm.at[src0.at[0]], buf_a, sem_a)

        @pl.when(n_w > 1)
        def _():
            pltpu.async_copy(g_hbm.at[src0.at[1]], buf_b, sem_b)

        @pl.when(W_CH < n_w)
        def _():
            load_idx(1, src1, dst1, sem_i1)

        plsc.subcore_barrier()

        for j in range(ch):
            wi, pos = divmod(j, W_CH)
            sv, dv, _ = sets[wi % 2]
            buf, sem = (buf_a, sem_a) if j % 2 == 0 else (buf_b, sem_b)

            if pos == 0 and wi >= 1 and (wi + 1) * W_CH < ch:
                svn, dvn, semn = sets[(wi + 1) % 2]

                @pl.when((wi + 1) * W_CH < n_w)
                def _(wi=wi, svn=svn, dvn=dvn, semn=semn):
                    load_idx(wi + 1, svn, dvn, semn)

            @pl.when(j < n_w)
            def _(sv=sv, dv=dv, pos=pos, buf=buf, sem=sem):
                pltpu.make_async_copy(g_hbm.at[sv.at[pos]], buf, sem).wait()
                pltpu.sync_copy(buf, acc.at[dv.at[pos]], add=True)

            if j + 2 < ch:
                wi2, pos2 = divmod(j + 2, W_CH)
                sv2, dv2, sem2 = sets[wi2 % 2]
                if pos2 == 0:
                    @pl.when(wi2 * W_CH < n_w)
                    def _(sv2=sv2, dv2=dv2, sem2=sem2):
                        wait_idx(sv2, dv2, sem2)

                @pl.when(j + 2 < n_w)
                def _(sv2=sv2, pos2=pos2, buf=buf, sem=sem):
                    pltpu.async_copy(g_hbm.at[sv2.at[pos2]], buf, sem)

        plsc.subcore_barrier()
        pltpu.sync_copy(acc.at[pl.ds(s * ZROWS, ZROWS)],
                        out_hbm.at[c].at[pl.ds(s * ZROWS, ZROWS)])

    return scatter_kernel


BM = 2000


def _first_tc(x_ref, w_ref, degp_ref, g_ref):
    deg = degp_ref[0] + degp_ref[1] + 1.0
    d = lax.rsqrt(deg)
    g_ref[...] = (x_ref[...] @ w_ref[...]) * d


def _mid_tc(s_ref, g_ref, w_ref, b_ref, degp_ref, o_ref):
    deg = degp_ref[0] + degp_ref[1] + 1.0
    d = lax.rsqrt(deg)
    z = d * (s_ref[0] + s_ref[1] + g_ref[...]) + b_ref[...]
    h = jnp.maximum(z, 0.0) @ w_ref[...]
    o_ref[...] = h * d


def _last_tc(s_ref, g_ref, b_ref, degp_ref, o_ref):
    deg = degp_ref[0] + degp_ref[1] + 1.0
    d = lax.rsqrt(deg)
    o_ref[...] = d * (s_ref[0] + s_ref[1] + g_ref[...]) + b_ref[...]


def _row_grid():
    return N // BM


_SPEC_S = pl.BlockSpec((2, BM, D), lambda i: (0, i, 0))
_SPEC_ROWS = pl.BlockSpec((BM, D), lambda i: (i, 0))
_SPEC_W = pl.BlockSpec((D, D), lambda i: (0, 0))
_SPEC_B = pl.BlockSpec((1, D), lambda i: (0, 0))
_SPEC_DEG = pl.BlockSpec((2, BM, 1), lambda i: (0, i, 0))


def kernel(x, edge_index, W1, b1, W2, b2, W3, b3):
    src = edge_index[0]
    dst = edge_index[1]
    e = src.shape[0]
    ch = -(-e // (NW * K))
    ch = ((ch + W_CH - 1) // W_CH) * W_CH
    e_pad = NW * ch * K
    pad = e_pad - e
    pad_dst = N + (jnp.arange(pad, dtype=jnp.int32) % (ACC_ROWS - N))
    src_p = jnp.concatenate([src, jnp.zeros((pad,), jnp.int32)]).reshape(NW, ch, K)
    dst_p = jnp.concatenate([dst, pad_dst]).reshape(NW, ch, K)

    zeros_deg = jnp.zeros((DEG_Z,), jnp.float32)

    degp = _make_deg_kernel(ch, e)(dst_p, zeros_deg)
    degp = degp[:, :N, None]

    scatter = _make_scatter_kernel(ch, e)

    first = pl.pallas_call(
        _first_tc,
        grid=(_row_grid(),),
        in_specs=[_SPEC_ROWS, _SPEC_W, _SPEC_DEG],
        out_specs=_SPEC_ROWS,
        out_shape=jax.ShapeDtypeStruct((N, D), jnp.float32),
    )
    mid = pl.pallas_call(
        _mid_tc,
        grid=(_row_grid(),),
        in_specs=[_SPEC_S, _SPEC_ROWS, _SPEC_W, _SPEC_B, _SPEC_DEG],
        out_specs=_SPEC_ROWS,
        out_shape=jax.ShapeDtypeStruct((N, D), jnp.float32),
    )
    last = pl.pallas_call(
        _last_tc,
        grid=(_row_grid(),),
        in_specs=[_SPEC_S, _SPEC_ROWS, _SPEC_B, _SPEC_DEG],
        out_specs=_SPEC_ROWS,
        out_shape=jax.ShapeDtypeStruct((N, D), jnp.float32),
    )

    g1 = first(x, W1, degp)
    s1 = scatter(g1, src_p, dst_p)
    g2 = mid(s1, g1, W2, b1.reshape(1, D), degp)
    s2 = scatter(g2, src_p, dst_p)
    g3 = mid(s2, g2, W3, b2.reshape(1, D), degp)
    s3 = scatter(g3, src_p, dst_p)
    return last(s3, g3, b3.reshape(1, D), degp)


# speedup vs baseline: 1.1171x; 1.0785x over previous
import functools

import jax
import jax.numpy as jnp
from jax import lax
from jax.experimental import pallas as pl
from jax.experimental.pallas import tpu as pltpu
from jax.experimental.pallas import tpu_sc as plsc

N = 10000
D = 128
NC = 2
NS = 16
NW = NC * NS
K = 96

ZROWS = 632
ACC_ROWS = NS * ZROWS

DEG_ROWS = 10240
DEG_Z = DEG_ROWS // NS


def _mesh():
    return plsc.VectorSubcoreMesh(core_axis_name="c", subcore_axis_name="s")


def _make_deg_kernel(ch, e):
    @functools.partial(
        pl.kernel,
        out_type=jax.ShapeDtypeStruct((NC, DEG_ROWS), jnp.float32),
        mesh=_mesh(),
        scratch_types=[
            pltpu.VMEM((ch, K), jnp.int32),
            pltpu.VMEM((K,), jnp.float32),
            pltpu.VMEM_SHARED((DEG_ROWS,), jnp.float32),
        ],
    )
    def deg_kernel(dst_hbm, zeros_hbm, out_hbm, dst_v, ones_v, acc):
        c = lax.axis_index("c")
        s = lax.axis_index("s")
        w = s * NC + c
        n_w = jnp.minimum(ch, jnp.maximum(0, (e - w * (ch * K) + K - 1) // K))
        pltpu.sync_copy(dst_hbm.at[w], dst_v)
        for i in range(K // 16):
            ones_v[pl.ds(i * 16, 16)] = jnp.ones((16,), jnp.float32)
        pltpu.sync_copy(zeros_hbm, acc.at[pl.ds(s * DEG_Z, DEG_Z)])
        plsc.subcore_barrier()

        def body(j, carry):
            pltpu.sync_copy(ones_v, acc.at[dst_v.at[j]], add=True)
            return carry

        lax.fori_loop(0, n_w, body, 0)
        plsc.subcore_barrier()
        pltpu.sync_copy(acc.at[pl.ds(s * DEG_Z, DEG_Z)],
                        out_hbm.at[c].at[pl.ds(s * DEG_Z, DEG_Z)])

    return deg_kernel


W_CH = 16


def _make_scatter_kernel(ch_arr, ch, e):
    @functools.partial(
        pl.kernel,
        out_type=jax.ShapeDtypeStruct((NC, ACC_ROWS, D), jnp.float32),
        mesh=_mesh(),
        scratch_types=[
            pltpu.VMEM((W_CH, K), jnp.int32),
            pltpu.VMEM((W_CH, K), jnp.int32),
            pltpu.VMEM((W_CH, K), jnp.int32),
            pltpu.VMEM((W_CH, K), jnp.int32),
            pltpu.VMEM((K, D), jnp.float32),
            pltpu.VMEM((K, D), jnp.float32),
            pltpu.VMEM((K, D), jnp.float32),
            pltpu.VMEM((32, D), jnp.float32),
            pltpu.VMEM_SHARED((ACC_ROWS, D), jnp.float32),
            pltpu.SemaphoreType.DMA,
            pltpu.SemaphoreType.DMA,
            pltpu.SemaphoreType.DMA,
            pltpu.SemaphoreType.DMA,
            pltpu.SemaphoreType.DMA,
        ],
    )
    def scatter_kernel(g_hbm, src_hbm, dst_hbm, out_hbm,
                       src0, dst0, src1, dst1, buf_a, buf_b, buf_c, zbuf, acc,
                       sem_a, sem_b, sem_c, sem_i0, sem_i1):
        c = lax.axis_index("c")
        s = lax.axis_index("s")
        w = s * NC + c
        n_w = jnp.minimum(ch, jnp.maximum(0, (e - w * (ch * K) + K - 1) // K))
        n_win = (n_w + W_CH - 1) // W_CH

        def load_idx(wi, sv, dv, sem):
            base = wi * W_CH
            pltpu.async_copy(src_hbm.at[w].at[pl.ds(base, W_CH)], sv, sem)
            pltpu.async_copy(dst_hbm.at[w].at[pl.ds(base, W_CH)], dv, sem)

        def wait_idx(sv, dv, sem):
            pltpu.make_async_copy(src_hbm.at[w].at[pl.ds(0, W_CH)], sv,
                                  sem).wait()
            pltpu.make_async_copy(dst_hbm.at[w].at[pl.ds(0, W_CH)], dv,
                                  sem).wait()

        sets = ((src0, dst0, sem_i0), (src1, dst1, sem_i1))

        @pl.when(n_w > 0)
        def _():
            load_idx(0, src0, dst0, sem_i0)
        for i in range(32):
            for k in range(D // 16):
                zbuf[i, pl.ds(k * 16, 16)] = jnp.zeros((16,), jnp.float32)
        for r in range(19):
            pltpu.sync_copy(zbuf, acc.at[pl.ds(s * ZROWS + r * 32, 32)])
        pltpu.sync_copy(zbuf.at[pl.ds(0, 24)],
                        acc.at[pl.ds(s * ZROWS + 608, 24)])

        @pl.when(n_w > 0)
        def _():
            wait_idx(src0, dst0, sem_i0)
            pltpu.async_copy(g_hbm.at[src0.at[0]], buf_a, sem_a)

        @pl.when(n_w > 1)
        def _():
            pltpu.async_copy(g_hbm.at[src0.at[1]], buf_b, sem_b)

        @pl.when(n_w > 2)
        def _():
            pltpu.async_copy(g_hbm.at[src0.at[2]], buf_c, sem_c)

        @pl.when(W_CH < n_w)
        def _():
            load_idx(1, src1, dst1, sem_i1)

        plsc.subcore_barrier()

        bufs = ((buf_a, sem_a), (buf_b, sem_b), (buf_c, sem_c))

        for j in range(ch):
            wi, pos = divmod(j, W_CH)
            sv, dv, _ = sets[wi % 2]
            buf, sem = bufs[j % 3]

            if pos == 0 and wi >= 1 and (wi + 1) * W_CH < ch:
                svn, dvn, semn = sets[(wi + 1) % 2]

                @pl.when((wi + 1) * W_CH < n_w)
                def _(wi=wi, svn=svn, dvn=dvn, semn=semn):
                    load_idx(wi + 1, svn, dvn, semn)

            @pl.when(j < n_w)
            def _(sv=sv, dv=dv, pos=pos, buf=buf, sem=sem):
                pltpu.make_async_copy(g_hbm.at[sv.at[pos]], buf, sem).wait()
                pltpu.sync_copy(buf, acc.at[dv.at[pos]], add=True)

            if j + 3 < ch:
                wi2, pos2 = divmod(j + 3, W_CH)
                sv2, dv2, sem2 = sets[wi2 % 2]
                if pos2 == 0:
                    @pl.when(wi2 * W_CH < n_w)
                    def _(sv2=sv2, dv2=dv2, sem2=sem2):
                        wait_idx(sv2, dv2, sem2)

                @pl.when(j + 3 < n_w)
                def _(sv2=sv2, pos2=pos2, buf=buf, sem=sem):
                    pltpu.async_copy(g_hbm.at[sv2.at[pos2]], buf, sem)

        plsc.subcore_barrier()
        pltpu.sync_copy(acc.at[pl.ds(s * ZROWS, ZROWS)],
                        out_hbm.at[c].at[pl.ds(s * ZROWS, ZROWS)])

    return scatter_kernel


BM = 2000


def _first_tc(x_ref, w_ref, degp_ref, g_ref):
    deg = degp_ref[0] + degp_ref[1] + 1.0
    d = lax.rsqrt(deg)
    g_ref[...] = (x_ref[...] @ w_ref[...]) * d


def _mid_tc(s_ref, g_ref, w_ref, b_ref, degp_ref, o_ref):
    deg = degp_ref[0] + degp_ref[1] + 1.0
    d = lax.rsqrt(deg)
    z = d * (s_ref[0] + s_ref[1] + g_ref[...]) + b_ref[...]
    h = jnp.maximum(z, 0.0) @ w_ref[...]
    o_ref[...] = h * d


def _last_tc(s_ref, g_ref, b_ref, degp_ref, o_ref):
    deg = degp_ref[0] + degp_ref[1] + 1.0
    d = lax.rsqrt(deg)
    o_ref[...] = d * (s_ref[0] + s_ref[1] + g_ref[...]) + b_ref[...]


def _row_grid():
    return N // BM


_SPEC_S = pl.BlockSpec((2, BM, D), lambda i: (0, i, 0))
_SPEC_ROWS = pl.BlockSpec((BM, D), lambda i: (i, 0))
_SPEC_W = pl.BlockSpec((D, D), lambda i: (0, 0))
_SPEC_B = pl.BlockSpec((1, D), lambda i: (0, 0))
_SPEC_DEG = pl.BlockSpec((2, BM, 1), lambda i: (0, i, 0))


def kernel(x, edge_index, W1, b1, W2, b2, W3, b3):
    src = edge_index[0]
    dst = edge_index[1]
    e = src.shape[0]
    ch = -(-e // (NW * K))
    ch_arr = ((ch + W_CH - 1) // W_CH) * W_CH
    e_pad = NW * ch * K
    pad = e_pad - e
    pad_dst = N + (jnp.arange(pad, dtype=jnp.int32) % (ACC_ROWS - N))
    src_p = jnp.concatenate([src, jnp.zeros((pad,), jnp.int32)]).reshape(NW, ch, K)
    dst_p = jnp.concatenate([dst, pad_dst]).reshape(NW, ch, K)
    junk = jnp.zeros((NW, ch_arr - ch, K), jnp.int32)
    src_a = jnp.concatenate([src_p, junk], axis=1)
    dst_a = jnp.concatenate([dst_p, junk + N], axis=1)

    zeros_deg = jnp.zeros((DEG_Z,), jnp.float32)

    degp = _make_deg_kernel(ch, e)(dst_p, zeros_deg)
    degp = degp[:, :N, None]

    scatter = _make_scatter_kernel(ch_arr, ch, e)

    first = pl.pallas_call(
        _first_tc,
        grid=(_row_grid(),),
        in_specs=[_SPEC_ROWS, _SPEC_W, _SPEC_DEG],
        out_specs=_SPEC_ROWS,
        out_shape=jax.ShapeDtypeStruct((N, D), jnp.float32),
    )
    mid = pl.pallas_call(
        _mid_tc,
        grid=(_row_grid(),),
        in_specs=[_SPEC_S, _SPEC_ROWS, _SPEC_W, _SPEC_B, _SPEC_DEG],
        out_specs=_SPEC_ROWS,
        out_shape=jax.ShapeDtypeStruct((N, D), jnp.float32),
    )
    last = pl.pallas_call(
        _last_tc,
        grid=(_row_grid(),),
        in_specs=[_SPEC_S, _SPEC_ROWS, _SPEC_B, _SPEC_DEG],
        out_specs=_SPEC_ROWS,
        out_shape=jax.ShapeDtypeStruct((N, D), jnp.float32),
    )

    g1 = first(x, W1, degp)
    s1 = scatter(g1, src_a, dst_a)
    g2 = mid(s1, g1, W2, b1.reshape(1, D), degp)
    s2 = scatter(g2, src_a, dst_a)
    g3 = mid(s2, g2, W3, b2.reshape(1, D), degp)
    s3 = scatter(g3, src_a, dst_a)
    return last(s3, g3, b3.reshape(1, D), degp)
